# baseline (speedup 1.0000x reference)
import jax
import jax.numpy as jnp
from jax.experimental import pallas as pl

_B = 4096
_S = 200
_D = 64
_SB = 20


def _tc_body(x_ref, o_ref):
    @pl.when(pl.program_id(0) == 0)
    def _init():
        o_ref[...] = jnp.zeros_like(o_ref)

    o_ref[...] += jnp.sum(x_ref[...], axis=0)


def kernel(inputs):
    x3 = jnp.transpose(inputs, (1, 2, 0))
    out_t = pl.pallas_call(
        _tc_body,
        grid=(_S // _SB,),
        in_specs=[pl.BlockSpec((_SB, _D, _B), lambda i: (i, 0, 0))],
        out_specs=pl.BlockSpec((_D, _B), lambda i: (0, 0)),
        out_shape=jax.ShapeDtypeStruct((_D, _B), jnp.float32),
    )(x3)
    return jnp.transpose(out_t, (1, 0))
